# baseline (speedup 1.0000x reference)
import functools

import jax
import jax.numpy as jnp
from jax import lax
from jax.experimental import pallas as pl
from jax.experimental.pallas import tpu as pltpu
from jax.experimental.pallas import tpu_sc as plsc

N_NODES = 10000
D = 128
E = 320000

NC = 2
NS = 16
NW = NC * NS

CHUNK = 128
CHUNKS_PER_W = 80
PHASES = (40, 40)
STAGE_ROWS = 40
E_PER_W = CHUNK * CHUNKS_PER_W
E_PAD = NW * E_PER_W

N_PAD = 10240
ROWS_PER_TILE = N_PAD // NS
ACC_ROWS = N_PAD


def _sc_body(x_hbm, src_hbm, dst_hbm, out_hbm, src_v, dst_v, r0, r1, acc,
             sem0, sem1):
    cid = lax.axis_index("c")
    sid = lax.axis_index("s")
    wid = cid * NS + sid

    zeros16 = jnp.zeros((16,), jnp.float32)

    def zrow(i, carry):
        for c in range(D // 16):
            r0[i, pl.ds(c * 16, 16)] = zeros16
        return carry

    lax.fori_loop(0, CHUNK, zrow, 0)

    base = sid * ROWS_PER_TILE
    for k in range(ROWS_PER_TILE // CHUNK):
        pltpu.sync_copy(r0, acc.at[pl.ds(base + k * CHUNK, CHUNK)])
    plsc.subcore_barrier()

    def gather_start(j, rbuf, sem):
        pltpu.async_copy(x_hbm.at[src_v.at[j]], rbuf, sem)

    def gather_wait(rbuf, sem):
        pltpu.make_async_copy(x_hbm.at[src_v.at[0]], rbuf, sem).wait()

    row0 = 0
    for nrows in PHASES:
        pltpu.sync_copy(src_hbm.at[wid, pl.ds(row0, nrows)], src_v)
        pltpu.sync_copy(dst_hbm.at[wid, pl.ds(row0, nrows)], dst_v)
        row0 += nrows
        gather_start(0, r0, sem0)

        def step(j, carry):
            c0 = 2 * j
            gather_start(c0 + 1, r1, sem1)
            gather_wait(r0, sem0)
            pltpu.sync_copy(r0, acc.at[dst_v.at[c0]], add=True)

            @pl.when(j < nrows // 2 - 1)
            def _():
                gather_start(c0 + 2, r0, sem0)

            gather_wait(r1, sem1)
            pltpu.sync_copy(r1, acc.at[dst_v.at[c0 + 1]], add=True)
            return carry

        lax.fori_loop(0, nrows // 2, step, 0)

    plsc.subcore_barrier()
    pltpu.sync_copy(acc.at[pl.ds(base, ROWS_PER_TILE)],
                    out_hbm.at[cid, pl.ds(base, ROWS_PER_TILE)])


def _sc_aggregate(x, src2, dst2):
    mesh = plsc.VectorSubcoreMesh(core_axis_name="c", subcore_axis_name="s")
    return pl.kernel(
        _sc_body,
        out_type=jax.ShapeDtypeStruct((NC, N_PAD, D), jnp.float32),
        mesh=mesh,
        scratch_types=[
            pltpu.VMEM((STAGE_ROWS, CHUNK), jnp.int32),
            pltpu.VMEM((STAGE_ROWS, CHUNK), jnp.int32),
            pltpu.VMEM((CHUNK, D), jnp.float32),
            pltpu.VMEM((CHUNK, D), jnp.float32),
            pltpu.VMEM_SHARED((ACC_ROWS, D), jnp.float32),
            pltpu.SemaphoreType.DMA,
            pltpu.SemaphoreType.DMA,
        ],
    )(x, src2, dst2)


BM = 2000


def _mm_body(p_ref, w_ref, b_ref, o_ref):
    agg = p_ref[0] + p_ref[1]
    o_ref[...] = (
        jnp.dot(agg, w_ref[...], preferred_element_type=jnp.float32)
        + b_ref[...]
    )


def _mm_call(partial, wt, b2):
    return pl.pallas_call(
        _mm_body,
        grid=(N_NODES // BM,),
        in_specs=[
            pl.BlockSpec((NC, BM, D), lambda i: (0, i, 0)),
            pl.BlockSpec((D, D), lambda i: (0, 0)),
            pl.BlockSpec((1, D), lambda i: (0, 0)),
        ],
        out_specs=pl.BlockSpec((BM, D), lambda i: (i, 0)),
        out_shape=jax.ShapeDtypeStruct((N_NODES, D), jnp.float32),
    )(partial, wt, b2)


@jax.jit
def _gcn(x, edge_index, W, b):
    src = edge_index[0].astype(jnp.int32)
    dst = edge_index[1].astype(jnp.int32)
    npad = E_PAD - E
    pad_src = jnp.arange(npad, dtype=jnp.int32) % N_NODES
    pad_dst = N_NODES + jnp.arange(npad, dtype=jnp.int32) % (N_PAD - N_NODES)
    src3 = jnp.concatenate([src, pad_src]).reshape(NW, CHUNKS_PER_W, CHUNK)
    dst3 = jnp.concatenate([dst, pad_dst]).reshape(NW, CHUNKS_PER_W, CHUNK)
    partial = _sc_aggregate(x, src3, dst3)
    return _mm_call(partial, W.T, b.reshape(1, D))


def kernel(x, edge_index, W, b):
    return _gcn(x, edge_index, W, b)


# speedup vs baseline: 1.0095x; 1.0095x over previous
import functools

import jax
import jax.numpy as jnp
from jax import lax
from jax.experimental import pallas as pl
from jax.experimental.pallas import tpu as pltpu
from jax.experimental.pallas import tpu_sc as plsc

N_NODES = 10000
D = 128
E = 320000

NC = 2
NS = 16
NW = NC * NS

CHUNK = 128
CHUNKS_PER_W = 80
PHASES = (40, 40)
STAGE_ROWS = 40
E_PER_W = CHUNK * CHUNKS_PER_W
E_PAD = NW * E_PER_W

N_PAD = 10240
ROWS_PER_TILE = N_PAD // NS
ACC_ROWS = N_PAD


def _sc_body(x_hbm, src_hbm, dst_hbm, out_hbm, src_v, dst_v, r0, r1, acc,
             sem0, sem1):
    cid = lax.axis_index("c")
    sid = lax.axis_index("s")
    wid = cid * NS + sid

    def gather_start(j, rbuf, sem):
        pltpu.async_copy(x_hbm.at[src_v.at[j]], rbuf, sem)

    def gather_wait(rbuf, sem):
        pltpu.make_async_copy(x_hbm.at[src_v.at[0]], rbuf, sem).wait()

    pltpu.sync_copy(src_hbm.at[wid, pl.ds(0, PHASES[0])], src_v)
    pltpu.sync_copy(dst_hbm.at[wid, pl.ds(0, PHASES[0])], dst_v)
    gather_start(0, r1, sem1)

    zeros16 = jnp.zeros((16,), jnp.float32)

    def zrow(i, carry):
        for c in range(D // 16):
            r0[i, pl.ds(c * 16, 16)] = zeros16
        return carry

    lax.fori_loop(0, CHUNK, zrow, 0)

    base = sid * ROWS_PER_TILE
    for k in range(ROWS_PER_TILE // CHUNK):
        pltpu.sync_copy(r0, acc.at[pl.ds(base + k * CHUNK, CHUNK)])
    plsc.subcore_barrier()

    def phase_loop(nrows, ra, sa, rb, sb):
        def step(j, carry):
            c0 = 2 * j
            gather_start(c0 + 1, rb, sb)
            gather_wait(ra, sa)
            pltpu.sync_copy(ra, acc.at[dst_v.at[c0]], add=True)

            @pl.when(j < nrows // 2 - 1)
            def _():
                gather_start(c0 + 2, ra, sa)

            gather_wait(rb, sb)
            pltpu.sync_copy(rb, acc.at[dst_v.at[c0 + 1]], add=True)
            return carry

        lax.fori_loop(0, nrows // 2, step, 0)

    phase_loop(PHASES[0], r1, sem1, r0, sem0)
    row0 = PHASES[0]
    for nrows in PHASES[1:]:
        pltpu.sync_copy(src_hbm.at[wid, pl.ds(row0, nrows)], src_v)
        pltpu.sync_copy(dst_hbm.at[wid, pl.ds(row0, nrows)], dst_v)
        row0 += nrows
        gather_start(0, r0, sem0)
        phase_loop(nrows, r0, sem0, r1, sem1)

    plsc.subcore_barrier()
    pltpu.sync_copy(acc.at[pl.ds(base, ROWS_PER_TILE)],
                    out_hbm.at[cid, pl.ds(base, ROWS_PER_TILE)])


def _sc_aggregate(x, src2, dst2):
    mesh = plsc.VectorSubcoreMesh(core_axis_name="c", subcore_axis_name="s")
    return pl.kernel(
        _sc_body,
        out_type=jax.ShapeDtypeStruct((NC, N_PAD, D), jnp.float32),
        mesh=mesh,
        scratch_types=[
            pltpu.VMEM((STAGE_ROWS, CHUNK), jnp.int32),
            pltpu.VMEM((STAGE_ROWS, CHUNK), jnp.int32),
            pltpu.VMEM((CHUNK, D), jnp.float32),
            pltpu.VMEM((CHUNK, D), jnp.float32),
            pltpu.VMEM_SHARED((ACC_ROWS, D), jnp.float32),
            pltpu.SemaphoreType.DMA,
            pltpu.SemaphoreType.DMA,
        ],
    )(x, src2, dst2)


BM = 2000


def _mm_body(p_ref, w_ref, b_ref, o_ref):
    agg = p_ref[0] + p_ref[1]
    o_ref[...] = (
        jnp.dot(agg, w_ref[...], preferred_element_type=jnp.float32)
        + b_ref[...]
    )


def _mm_call(partial, wt, b2):
    return pl.pallas_call(
        _mm_body,
        grid=(N_NODES // BM,),
        in_specs=[
            pl.BlockSpec((NC, BM, D), lambda i: (0, i, 0)),
            pl.BlockSpec((D, D), lambda i: (0, 0)),
            pl.BlockSpec((1, D), lambda i: (0, 0)),
        ],
        out_specs=pl.BlockSpec((BM, D), lambda i: (i, 0)),
        out_shape=jax.ShapeDtypeStruct((N_NODES, D), jnp.float32),
    )(partial, wt, b2)


@jax.jit
def _gcn(x, edge_index, W, b):
    src = edge_index[0].astype(jnp.int32)
    dst = edge_index[1].astype(jnp.int32)
    npad = E_PAD - E
    pad_src = jnp.arange(npad, dtype=jnp.int32) % N_NODES
    pad_dst = N_NODES + jnp.arange(npad, dtype=jnp.int32) % (N_PAD - N_NODES)
    src3 = jnp.concatenate([src, pad_src]).reshape(NW, CHUNKS_PER_W, CHUNK)
    dst3 = jnp.concatenate([dst, pad_dst]).reshape(NW, CHUNKS_PER_W, CHUNK)
    partial = _sc_aggregate(x, src3, dst3)
    return _mm_call(partial, W.T, b.reshape(1, D))


def kernel(x, edge_index, W, b):
    return _gcn(x, edge_index, W, b)


# speedup vs baseline: 1.0134x; 1.0039x over previous
import jax
import jax.numpy as jnp
from jax import lax
from jax.experimental import pallas as pl
from jax.experimental.pallas import tpu as pltpu
from jax.experimental.pallas import tpu_sc as plsc

N_NODES = 10000
D = 128
E = 320000

NC = 2
NS = 16
NW = NC * NS

CHUNK = 128
CHUNKS_PER_W = 80
PHASES = (40, 40)
STAGE_ROWS = 40
E_PER_W = CHUNK * CHUNKS_PER_W
E_PAD = NW * E_PER_W

N_PAD = 10240
ROWS_PER_TILE = N_PAD // NS
ACC_ROWS = N_PAD


def _sc_body(x_hbm, src_hbm, dst_hbm, out_hbm, src_v, dst_v, r0, r1, acc,
             sem0, sem1):
    cid = lax.axis_index("c")
    sid = lax.axis_index("s")
    wid = cid * NS + sid

    def gather_start(j, rbuf, sem):
        pltpu.async_copy(x_hbm.at[src_v.at[j]], rbuf, sem)

    def gather_wait(rbuf, sem):
        pltpu.make_async_copy(x_hbm.at[src_v.at[0]], rbuf, sem).wait()

    pltpu.sync_copy(src_hbm.at[wid, pl.ds(0, PHASES[0])], src_v)
    pltpu.sync_copy(dst_hbm.at[wid, pl.ds(0, PHASES[0])], dst_v)
    gather_start(0, r1, sem1)

    zeros16 = jnp.zeros((16,), jnp.float32)

    def zrow(i, carry):
        for c in range(D // 16):
            r0[i, pl.ds(c * 16, 16)] = zeros16
        return carry

    lax.fori_loop(0, CHUNK, zrow, 0)

    base = sid * ROWS_PER_TILE
    for k in range(ROWS_PER_TILE // CHUNK):
        pltpu.sync_copy(r0, acc.at[pl.ds(base + k * CHUNK, CHUNK)])
    plsc.subcore_barrier()

    def phase_loop(nrows, ra, sa, rb, sb):
        def step(j, carry):
            c0 = 2 * j
            gather_start(c0 + 1, rb, sb)
            gather_wait(ra, sa)
            pltpu.sync_copy(ra, acc.at[dst_v.at[c0]], add=True)

            @pl.when(j < nrows // 2 - 1)
            def _():
                gather_start(c0 + 2, ra, sa)

            gather_wait(rb, sb)
            pltpu.sync_copy(rb, acc.at[dst_v.at[c0 + 1]], add=True)
            return carry

        lax.fori_loop(0, nrows // 2, step, 0)

    phase_loop(PHASES[0], r1, sem1, r0, sem0)
    row0 = PHASES[0]
    for nrows in PHASES[1:]:
        pltpu.sync_copy(src_hbm.at[wid, pl.ds(row0, nrows)], src_v)
        pltpu.sync_copy(dst_hbm.at[wid, pl.ds(row0, nrows)], dst_v)
        row0 += nrows
        gather_start(0, r0, sem0)
        phase_loop(nrows, r0, sem0, r1, sem1)

    plsc.subcore_barrier()
    pltpu.sync_copy(acc.at[pl.ds(base, ROWS_PER_TILE)],
                    out_hbm.at[cid, pl.ds(base, ROWS_PER_TILE)])


def _sc_aggregate(x, src2, dst2):
    mesh = plsc.VectorSubcoreMesh(core_axis_name="c", subcore_axis_name="s")
    return pl.kernel(
        _sc_body,
        out_type=jax.ShapeDtypeStruct((NC, N_PAD, D), jnp.float32),
        mesh=mesh,
        scratch_types=[
            pltpu.VMEM((STAGE_ROWS, CHUNK), jnp.int32),
            pltpu.VMEM((STAGE_ROWS, CHUNK), jnp.int32),
            pltpu.VMEM((CHUNK, D), jnp.float32),
            pltpu.VMEM((CHUNK, D), jnp.float32),
            pltpu.VMEM_SHARED((ACC_ROWS, D), jnp.float32),
            pltpu.SemaphoreType.DMA,
            pltpu.SemaphoreType.DMA,
        ],
    )(x, src2, dst2)


BM = 2000


def _mm_body(p_ref, w_ref, b_ref, o_ref):
    agg = p_ref[0] + p_ref[1]
    o_ref[...] = (
        jnp.dot(agg, w_ref[...], preferred_element_type=jnp.float32)
        + b_ref[...]
    )


def _mm_call(partial, wt, b2):
    return pl.pallas_call(
        _mm_body,
        grid=(N_NODES // BM,),
        in_specs=[
            pl.BlockSpec((NC, BM, D), lambda i: (0, i, 0)),
            pl.BlockSpec((D, D), lambda i: (0, 0)),
            pl.BlockSpec((1, D), lambda i: (0, 0)),
        ],
        out_specs=pl.BlockSpec((BM, D), lambda i: (i, 0)),
        out_shape=jax.ShapeDtypeStruct((N_NODES, D), jnp.float32),
    )(partial, wt, b2)


@jax.jit
def _gcn(x, edge_index, W, b):
    src = edge_index[0].astype(jnp.int32)
    dst = edge_index[1].astype(jnp.int32)
    npad = E_PAD - E
    pad_src = jnp.arange(npad, dtype=jnp.int32) % N_NODES
    pad_dst = N_NODES + jnp.arange(npad, dtype=jnp.int32) % (N_PAD - N_NODES)
    src3 = jnp.concatenate([src, pad_src]).reshape(NW, CHUNKS_PER_W, CHUNK)
    dst3 = jnp.concatenate([dst, pad_dst]).reshape(NW, CHUNKS_PER_W, CHUNK)
    partial = _sc_aggregate(x, src3, dst3)
    return _mm_call(partial, W.T, b.reshape(1, D))


def kernel(x, edge_index, W, b):
    return _gcn(x, edge_index, W, b)
